# baseline (speedup 1.0000x reference)
import functools

import jax
import jax.numpy as jnp
from jax import lax
from jax.experimental import pallas as pl
from jax.experimental.pallas import tpu as pltpu
from jax.experimental.pallas import tpu_sc as plsc

N_ENTITIES = 1000000
N_RELATIONS = 1000
C = 64
T = 16384

NC = 2
NS = 16
L = 16
NW = NC * NS
TPW = T // NW
CH = 128
NCHUNK = TPW // CH

_mesh = plsc.VectorSubcoreMesh(core_axis_name="c", subcore_axis_name="s")


@functools.partial(
    pl.kernel,
    mesh=_mesh,
    compiler_params=pltpu.CompilerParams(needs_layout_passes=False),
    out_type=jax.ShapeDtypeStruct((T,), jnp.float32),
    scratch_types=[
        pltpu.VMEM((NCHUNK, CH), jnp.int32),
        pltpu.VMEM((NCHUNK, CH), jnp.int32),
        pltpu.VMEM((NCHUNK, CH), jnp.int32),
        pltpu.VMEM((2, CH, C), jnp.float32),
        pltpu.VMEM((2, CH, C), jnp.float32),
        pltpu.VMEM((2, CH, C), jnp.float32),
        pltpu.VMEM((TPW,), jnp.float32),
        pltpu.SemaphoreType.DMA,
        pltpu.SemaphoreType.DMA,
    ],
)
def _distmult_sc(ent_hbm, rel_hbm, s_hbm, r_hbm, o_hbm, out_hbm,
                 sidx, ridx, oidx, es_v, wr_v, eo_v, out_v, sem0, sem1):
    wid = lax.axis_index("s") * NC + lax.axis_index("c")
    base = wid * TPW

    pltpu.sync_copy(s_hbm.at[wid], sidx)
    pltpu.sync_copy(r_hbm.at[wid], ridx)
    pltpu.sync_copy(o_hbm.at[wid], oidx)

    sems = (sem0, sem1)

    def fire_chunk(j, slot, sem):
        def fire_group(g, carry):
            sv = sidx[j, pl.ds(g * L, L)]
            rv = ridx[j, pl.ds(g * L, L)]
            ov = oidx[j, pl.ds(g * L, L)]
            t0 = g * L
            for tloc in range(L):
                pltpu.async_copy(
                    ent_hbm.at[pl.ds(sv[tloc], 1), :],
                    es_v.at[slot, pl.ds(t0 + tloc, 1), :], sem)
                pltpu.async_copy(
                    rel_hbm.at[pl.ds(rv[tloc], 1), :],
                    wr_v.at[slot, pl.ds(t0 + tloc, 1), :], sem)
                pltpu.async_copy(
                    ent_hbm.at[pl.ds(ov[tloc], 1), :],
                    eo_v.at[slot, pl.ds(t0 + tloc, 1), :], sem)
            return carry

        lax.fori_loop(0, CH // L, fire_group, 0)

    def drain_chunk(slot, sem):
        pltpu.make_async_copy(
            ent_hbm.at[pl.ds(0, CH), :], es_v.at[slot], sem).wait()
        pltpu.make_async_copy(
            rel_hbm.at[pl.ds(0, CH), :], wr_v.at[slot], sem).wait()
        pltpu.make_async_copy(
            ent_hbm.at[pl.ds(0, CH), :], eo_v.at[slot], sem).wait()

    lanes = lax.iota(jnp.int32, L)

    def compute_chunk(j, slot):
        def group_body(g, carry2):
            sums = jnp.zeros((L,), jnp.float32)
            for tloc in range(L):
                t = g * L + tloc
                acc = jnp.zeros((L,), jnp.float32)
                for k in range(C // L):
                    a = es_v[slot, t, pl.ds(k * L, L)]
                    b = wr_v[slot, t, pl.ds(k * L, L)]
                    d = eo_v[slot, t, pl.ds(k * L, L)]
                    acc = acc + a * b * d
                sums = jnp.where(lanes == tloc, jnp.sum(acc), sums)
            out_v[pl.ds(j * CH + g * L, L)] = sums
            return carry2

        lax.fori_loop(0, CH // L, group_body, 0)

    fire_chunk(0, 0, sems[0])
    for j in range(NCHUNK):
        nxt = j + 1
        if nxt < NCHUNK:
            fire_chunk(nxt, nxt % 2, sems[nxt % 2])
        drain_chunk(j % 2, sems[j % 2])
        compute_chunk(j, j % 2)

    pltpu.sync_copy(out_v, out_hbm.at[pl.ds(base, TPW)])


def kernel(initializations, rel_weights, sro_triples):
    s = sro_triples[0].reshape(NW, NCHUNK, CH)
    r = sro_triples[1].reshape(NW, NCHUNK, CH)
    o = sro_triples[2].reshape(NW, NCHUNK, CH)
    scores = _distmult_sc(initializations, rel_weights, s, r, o)
    dummy = jnp.take(initializations, sro_triples[0], axis=0)
    return scores + 0.0 * dummy[:, 0]


# speedup vs baseline: 1.0471x; 1.0471x over previous
import functools

import jax
import jax.numpy as jnp
from jax import lax
from jax.experimental import pallas as pl
from jax.experimental.pallas import tpu as pltpu
from jax.experimental.pallas import tpu_sc as plsc

N_ENTITIES = 1000000
N_RELATIONS = 1000
C = 64
T = 16384

NC = 2
NS = 16
L = 16
NW = NC * NS
TPW = T // NW
CH = 128
NCHUNK = TPW // CH

_mesh = plsc.VectorSubcoreMesh(core_axis_name="c", subcore_axis_name="s")


@functools.partial(
    pl.kernel,
    mesh=_mesh,
    compiler_params=pltpu.CompilerParams(needs_layout_passes=False),
    out_type=jax.ShapeDtypeStruct((T,), jnp.float32),
    scratch_types=[
        pltpu.VMEM((NCHUNK, CH), jnp.int32),
        pltpu.VMEM((NCHUNK, CH), jnp.int32),
        pltpu.VMEM((NCHUNK, CH), jnp.int32),
        pltpu.VMEM((2, CH, C), jnp.float32),
        pltpu.VMEM((2, CH, C), jnp.float32),
        pltpu.VMEM((2, CH, C), jnp.float32),
        pltpu.VMEM((TPW,), jnp.float32),
        pltpu.SemaphoreType.DMA,
        pltpu.SemaphoreType.DMA,
    ],
)
def _distmult_sc(ent_hbm, rel_hbm, s_hbm, r_hbm, o_hbm, out_hbm,
                 sidx, ridx, oidx, es_v, wr_v, eo_v, out_v, sem0, sem1):
    wid = lax.axis_index("s") * NC + lax.axis_index("c")
    base = wid * TPW

    pltpu.sync_copy(s_hbm.at[wid], sidx)
    pltpu.sync_copy(r_hbm.at[wid], ridx)
    pltpu.sync_copy(o_hbm.at[wid], oidx)

    sems = (sem0, sem1)

    def fire_chunk(j, slot, sem):
        def fire_group(g, carry):
            sv = sidx[j, pl.ds(g * L, L)]
            rv = ridx[j, pl.ds(g * L, L)]
            ov = oidx[j, pl.ds(g * L, L)]
            t0 = g * L
            for tloc in range(L):
                pltpu.async_copy(
                    ent_hbm.at[pl.ds(sv[tloc], 1), :],
                    es_v.at[slot, pl.ds(t0 + tloc, 1), :], sem)
                pltpu.async_copy(
                    rel_hbm.at[pl.ds(rv[tloc], 1), :],
                    wr_v.at[slot, pl.ds(t0 + tloc, 1), :], sem)
                pltpu.async_copy(
                    ent_hbm.at[pl.ds(ov[tloc], 1), :],
                    eo_v.at[slot, pl.ds(t0 + tloc, 1), :], sem)
            return carry

        lax.fori_loop(0, CH // L, fire_group, 0)

    def drain_chunk(slot, sem):
        pltpu.make_async_copy(
            ent_hbm.at[pl.ds(0, CH), :], es_v.at[slot], sem).wait()
        pltpu.make_async_copy(
            rel_hbm.at[pl.ds(0, CH), :], wr_v.at[slot], sem).wait()
        pltpu.make_async_copy(
            ent_hbm.at[pl.ds(0, CH), :], eo_v.at[slot], sem).wait()

    lanes = lax.iota(jnp.int32, L)

    def compute_chunk(j, slot):
        def group_body(g, carry2):
            sums = jnp.zeros((L,), jnp.float32)
            for tloc in range(L):
                t = g * L + tloc
                acc = jnp.zeros((L,), jnp.float32)
                for k in range(C // L):
                    a = es_v[slot, t, pl.ds(k * L, L)]
                    b = wr_v[slot, t, pl.ds(k * L, L)]
                    d = eo_v[slot, t, pl.ds(k * L, L)]
                    acc = acc + a * b * d
                sums = jnp.where(lanes == tloc, jnp.sum(acc), sums)
            out_v[pl.ds(j * CH + g * L, L)] = sums
            return carry2

        lax.fori_loop(0, CH // L, group_body, 0)

    fire_chunk(0, 0, sems[0])
    for j in range(NCHUNK):
        nxt = j + 1
        if nxt < NCHUNK:
            fire_chunk(nxt, nxt % 2, sems[nxt % 2])
        drain_chunk(j % 2, sems[j % 2])
        compute_chunk(j, j % 2)

    pltpu.sync_copy(out_v, out_hbm.at[pl.ds(base, TPW)])


def kernel(initializations, rel_weights, sro_triples):
    s = sro_triples[0].reshape(NW, NCHUNK, CH)
    r = sro_triples[1].reshape(NW, NCHUNK, CH)
    o = sro_triples[2].reshape(NW, NCHUNK, CH)
    return _distmult_sc(initializations, rel_weights, s, r, o)
